# baseline (speedup 1.0000x reference)
import functools

import jax
import jax.numpy as jnp
from jax import lax
from jax.experimental import pallas as pl
from jax.experimental.pallas import tpu as pltpu
from jax.experimental.pallas import tpu_sc as plsc

_EPS = 1e-12
_L = 16
_NC = 2
_NS = 16
_NW = _NC * _NS
_R = 32


def _tc_body(xc_ref, xd_ref, wb_ref, bb_ref, wo_ref, bo_ref,
             c_ref, d_ref, a_ref):
    xc = xc_ref[...]
    xd = xd_ref[...]
    wb = wb_ref[...]
    bb = bb_ref[...]
    wo = wo_ref[...]
    bo = bo_ref[...]

    t_c = xc[:, 0:1]
    t_d = xd[:, 0:1]
    lo = t_c - t_d
    hi = t_c + t_d
    frac = jnp.minimum(1.0, (0.0 - lo) / ((hi - lo) + _EPS))
    a1 = jnp.where(hi <= 0.0, 1.0, jnp.where(lo > 0.0, 0.0, frac))
    a2 = 1.0 - a1

    upper_l = jnp.minimum(hi, 0.0)
    cL = (lo + upper_l) * 0.5
    dL = (upper_l - lo) * 0.5
    lower_r = jnp.maximum(lo, 0.0)
    cR = (lower_r + hi) * 0.5
    dR = (hi - lower_r) * 0.5

    col = lax.broadcasted_iota(jnp.int32, xc.shape, 1)
    is0 = col == 0
    xl_c = jnp.where(is0, cL, xc)
    xl_d = jnp.where(is0, dL, xd)
    xr_c = jnp.where(is0, cR, xc)

    c1 = xl_c * wb + bb
    d1 = xl_d * jnp.abs(wb)
    c2 = xr_c * wo + bo

    amax = jnp.maximum(a1, a2)
    rcm = 1.0 / (amax + _EPS)
    ap1 = a1 * rcm
    ap2 = a2 * rcm
    c_out = (a1 * c1 + a2 * c2) / (a1 + a2 + _EPS)
    nc1 = ap1 * c1 + (1.0 - ap1) * c_out
    nc2 = ap2 * c2 + (1.0 - ap2) * c_out
    nd1 = ap1 * d1
    nd2 = ap2 * c2
    nl = jnp.minimum(nc1 - nd1, nc2 - nd2)
    nr = jnp.maximum(nc1 + nd1, nc2 + nd2)

    c_ref[...] = (nl + nr) * 0.5
    d_ref[...] = (nr - nl) * 0.5
    a_ref[...] = jnp.minimum(1.0, a1 + a2)


def _tc_call(x_c, x_delta, wb, bb, wo, bo):
    n, d = x_c.shape
    br = 1024
    wb2 = wb.reshape(1, d)
    bb2 = bb.reshape(1, d)
    wo2 = wo.reshape(1, d)
    bo2 = bo.reshape(1, d)
    row_spec = pl.BlockSpec((br, d), lambda i: (i, 0))
    vec_spec = pl.BlockSpec((1, d), lambda i: (0, 0))
    return pl.pallas_call(
        _tc_body,
        grid=(n // br,),
        in_specs=[row_spec, row_spec, vec_spec, vec_spec, vec_spec, vec_spec],
        out_specs=[row_spec, row_spec, pl.BlockSpec((br, 1), lambda i: (i, 0))],
        out_shape=[
            jax.ShapeDtypeStruct((n, d), jnp.float32),
            jax.ShapeDtypeStruct((n, d), jnp.float32),
            jax.ShapeDtypeStruct((n, 1), jnp.float32),
        ],
    )(x_c, x_delta, wb2, bb2, wo2, bo2)


def _sc_call(x_c, x_delta, wb, bb, wo, bo):
    n, d = x_c.shape
    rows_per_w = n // _NW
    chunks = rows_per_w // _R
    pairs = chunks // 2
    jvec = d // _L

    mesh = plsc.VectorSubcoreMesh(core_axis_name="c", subcore_axis_name="s")

    @functools.partial(
        pl.kernel,
        mesh=mesh,
        out_type=[
            jax.ShapeDtypeStruct((n, d), jnp.float32),
            jax.ShapeDtypeStruct((n, d), jnp.float32),
            jax.ShapeDtypeStruct((n,), jnp.float32),
        ],
        scratch_types=[
            pltpu.VMEM((2, _R, d), jnp.float32),
            pltpu.VMEM((2, _R, d), jnp.float32),
            pltpu.VMEM((2, _R, d), jnp.float32),
            pltpu.VMEM((2, _R, d), jnp.float32),
            pltpu.VMEM((2, _R), jnp.float32),
            pltpu.VMEM((d,), jnp.float32),
            pltpu.VMEM((d,), jnp.float32),
            pltpu.VMEM((d,), jnp.float32),
            pltpu.VMEM((d,), jnp.float32),
            pltpu.VMEM((d,), jnp.float32),
            pltpu.SemaphoreType.DMA,
            pltpu.SemaphoreType.DMA,
            pltpu.SemaphoreType.DMA,
            pltpu.SemaphoreType.DMA,
        ],
    )
    def k(xc_hbm, xd_hbm, wb_hbm, bb_hbm, wo_hbm, bo_hbm,
          oc_hbm, od_hbm, oa_hbm,
          xc_v, xd_v, oc_v, od_v, oa_v,
          wb_v, bb_v, wo_v, bo_v, awb_v,
          in_s0, in_s1, out_s0, out_s1):
        wid = lax.axis_index("s") * _NC + lax.axis_index("c")
        base = wid * rows_per_w
        in_sems = (in_s0, in_s1)
        out_sems = (out_s0, out_s1)

        pltpu.sync_copy(wb_hbm, wb_v)
        pltpu.sync_copy(bb_hbm, bb_v)
        pltpu.sync_copy(wo_hbm, wo_v)
        pltpu.sync_copy(bo_hbm, bo_v)
        for j in range(jvec):
            sl = pl.ds(j * _L, _L)
            awb_v[sl] = jnp.abs(wb_v[sl])
        head = pl.ds(0, _L)
        wb0 = wb_v[head][0]
        bb0 = bb_v[head][0]
        wo0 = wo_v[head][0]
        bo0 = bo_v[head][0]
        awb0 = awb_v[head][0]
        lane = lax.iota(jnp.int32, _L)
        mask0 = lane == 0

        def start_in(s, c):
            cb = base + c * _R
            pltpu.async_copy(xc_hbm.at[pl.ds(cb, _R)], xc_v.at[s], in_sems[s])
            pltpu.async_copy(xd_hbm.at[pl.ds(cb, _R)], xd_v.at[s], in_sems[s])

        def wait_in(s, c):
            cb = base + c * _R
            pltpu.make_async_copy(
                xc_hbm.at[pl.ds(cb, _R)], xc_v.at[s], in_sems[s]).wait()
            pltpu.make_async_copy(
                xd_hbm.at[pl.ds(cb, _R)], xd_v.at[s], in_sems[s]).wait()

        def start_out(s, c):
            cb = base + c * _R
            pltpu.async_copy(oc_v.at[s], oc_hbm.at[pl.ds(cb, _R)], out_sems[s])
            pltpu.async_copy(od_v.at[s], od_hbm.at[pl.ds(cb, _R)], out_sems[s])
            pltpu.async_copy(oa_v.at[s], oa_hbm.at[pl.ds(cb, _R)], out_sems[s])

        def wait_out(s, c):
            cb = base + c * _R
            pltpu.make_async_copy(
                oc_v.at[s], oc_hbm.at[pl.ds(cb, _R)], out_sems[s]).wait()
            pltpu.make_async_copy(
                od_v.at[s], od_hbm.at[pl.ds(cb, _R)], out_sems[s]).wait()
            pltpu.make_async_copy(
                oa_v.at[s], oa_hbm.at[pl.ds(cb, _R)], out_sems[s]).wait()

        def row_coeffs(s, r):
            xcv0 = xc_v[s, r, head]
            xdv0 = xd_v[s, r, head]
            t_c = jnp.full((_L,), xcv0[0])
            t_d = jnp.full((_L,), xdv0[0])
            lo = t_c - t_d
            hi = t_c + t_d
            frac = jnp.minimum(1.0, (0.0 - lo) / ((hi - lo) + _EPS))
            a1 = jnp.where(hi <= 0.0, 1.0, jnp.where(lo > 0.0, 0.0, frac))
            a2 = 1.0 - a1
            amax = jnp.maximum(a1, a2)
            rcm = 1.0 / (amax + _EPS)
            ap1 = a1 * rcm
            ap2 = a2 * rcm
            sinv = 1.0 / (a1 + a2 + _EPS)
            u1 = a1 * sinv
            q11 = (ap1 + (1.0 - ap1) * u1) * 0.5
            q21 = ((1.0 - ap2) * u1) * 0.5
            ap1h = ap1 * 0.5
            ap2h = ap2 * 0.5

            upper_l = jnp.minimum(hi, 0.0)
            cL = (lo + upper_l) * 0.5
            dL = (upper_l - lo) * 0.5
            lower_r = jnp.maximum(lo, 0.0)
            cR = (lower_r + hi) * 0.5
            c1_0 = cL * wb0 + bb0
            d1_0 = dL * awb0
            c2_0 = cR * wo0 + bo0
            e0 = c1_0 - c2_0
            g1_0 = ap1h * d1_0
            g2_0 = ap2h * c2_0
            mn0 = jnp.minimum(q11 * e0 - g1_0, q21 * e0 - g2_0)
            mx0 = jnp.maximum(q11 * e0 + g1_0, q21 * e0 + g2_0)
            oc0 = c2_0 + mn0 + mx0
            od0 = mx0 - mn0
            alpha = jnp.minimum(1.0, a1 + a2)
            return (q11, q21, ap1h, ap2h, oc0, od0, alpha)

        def compute_chunk(s):
            z = jnp.zeros((_L,), jnp.float32)

            def do_row(r, accs):
                acc0, acc1 = accs
                q11, q21, ap1h, ap2h, oc0, od0, alpha = row_coeffs(s, r)
                acc0 = jnp.where(lane == r, alpha, acc0)
                acc1 = jnp.where(lane == (r - _L), alpha, acc1)
                gb = 16
                for b in range(jvec // gb):
                    ins = []
                    for q in range(gb):
                        wsl = pl.ds((b * gb + q) * _L, _L)
                        ins.append((wsl, xc_v[s, r, wsl], xd_v[s, r, wsl],
                                    wb_v[wsl], bb_v[wsl], wo_v[wsl],
                                    bo_v[wsl], awb_v[wsl]))
                    outs = []
                    for q, (wsl, xcv, xdv, wbj, bbj, woj, boj, awbj) \
                            in enumerate(ins):
                        c1 = xcv * wbj + bbj
                        c2 = xcv * woj + boj
                        e = c1 - c2
                        g1 = xdv * awbj * ap1h
                        g2 = ap2h * c2
                        t1 = q11 * e
                        t2 = q21 * e
                        mn = jnp.minimum(t1 - g1, t2 - g2)
                        mx = jnp.maximum(t1 + g1, t2 + g2)
                        occ = c2 + mn + mx
                        odd = mx - mn
                        if b == 0 and q == 0:
                            occ = jnp.where(mask0, oc0, occ)
                            odd = jnp.where(mask0, od0, odd)
                        outs.append((wsl, occ, odd))
                    for wsl, occ, odd in outs:
                        oc_v[s, r, wsl] = occ
                        od_v[s, r, wsl] = odd
                return (acc0, acc1)

            acc0, acc1 = plsc.parallel_loop(
                0, _R, unroll=2, carry=(z, z))(do_row)
            oa_v[s, pl.ds(0, _L)] = acc0
            oa_v[s, pl.ds(_L, _L)] = acc1

        start_in(0, 0)
        start_in(1, 1)

        def do_pair_of_chunks(g2, carry):
            for s in (0, 1):
                c = g2 * 2 + s
                wait_in(s, c)

                @pl.when(g2 > 0)
                def _():
                    wait_out(s, c - 2)

                compute_chunk(s)
                start_out(s, c)

                @pl.when(c + 2 < chunks)
                def _():
                    start_in(s, c + 2)
            return carry

        lax.fori_loop(0, pairs, do_pair_of_chunks, 0)
        wait_out(0, chunks - 2)
        wait_out(1, chunks - 1)

    return k(x_c, x_delta, wb, bb, wo, bo)


_SC_ROWS = 14336


def kernel(x_c, x_delta, w_body, b_body, w_orelse, b_orelse):
    n, d = x_c.shape
    k = _SC_ROWS
    sc_oc, sc_od, sc_oa = _sc_call(
        x_c[:k], x_delta[:k], w_body, b_body, w_orelse, b_orelse)
    tc_oc, tc_od, tc_oa = _tc_call(
        x_c[k:], x_delta[k:], w_body, b_body, w_orelse, b_orelse)
    oc = jnp.concatenate([sc_oc, tc_oc], axis=0)
    od = jnp.concatenate([sc_od, tc_od], axis=0)
    oa = jnp.concatenate([sc_oa.reshape(k, 1), tc_oa], axis=0)
    return oc, od, oa


# speedup vs baseline: 1.6135x; 1.6135x over previous
import functools

import jax
import jax.numpy as jnp
from jax import lax
from jax.experimental import pallas as pl
from jax.experimental.pallas import tpu as pltpu
from jax.experimental.pallas import tpu_sc as plsc

_EPS = 1e-12
_L = 16
_NC = 2
_NS = 16
_NW = _NC * _NS
_R = 32


def _tc_body(xc_ref, xd_ref, wb_ref, bb_ref, wo_ref, bo_ref,
             c_ref, d_ref, a_ref):
    xc = xc_ref[...]
    xd = xd_ref[...]
    wb = wb_ref[...]
    bb = bb_ref[...]
    wo = wo_ref[...]
    bo = bo_ref[...]

    t_c = xc[:, 0:1]
    t_d = xd[:, 0:1]
    lo = t_c - t_d
    hi = t_c + t_d
    frac = jnp.minimum(1.0, (0.0 - lo) / ((hi - lo) + _EPS))
    a1 = jnp.where(hi <= 0.0, 1.0, jnp.where(lo > 0.0, 0.0, frac))
    a2 = 1.0 - a1

    upper_l = jnp.minimum(hi, 0.0)
    cL = (lo + upper_l) * 0.5
    dL = (upper_l - lo) * 0.5
    lower_r = jnp.maximum(lo, 0.0)
    cR = (lower_r + hi) * 0.5
    dR = (hi - lower_r) * 0.5

    col = lax.broadcasted_iota(jnp.int32, xc.shape, 1)
    is0 = col == 0
    xl_c = jnp.where(is0, cL, xc)
    xl_d = jnp.where(is0, dL, xd)
    xr_c = jnp.where(is0, cR, xc)

    c1 = xl_c * wb + bb
    d1 = xl_d * jnp.abs(wb)
    c2 = xr_c * wo + bo

    amax = jnp.maximum(a1, a2)
    rcm = 1.0 / (amax + _EPS)
    ap1 = a1 * rcm
    ap2 = a2 * rcm
    c_out = (a1 * c1 + a2 * c2) / (a1 + a2 + _EPS)
    nc1 = ap1 * c1 + (1.0 - ap1) * c_out
    nc2 = ap2 * c2 + (1.0 - ap2) * c_out
    nd1 = ap1 * d1
    nd2 = ap2 * c2
    nl = jnp.minimum(nc1 - nd1, nc2 - nd2)
    nr = jnp.maximum(nc1 + nd1, nc2 + nd2)

    c_ref[...] = (nl + nr) * 0.5
    d_ref[...] = (nr - nl) * 0.5
    a_ref[...] = jnp.minimum(1.0, a1 + a2)


def _tc_call(x_c, x_delta, wb, bb, wo, bo):
    n, d = x_c.shape
    br = 1024
    wb2 = wb.reshape(1, d)
    bb2 = bb.reshape(1, d)
    wo2 = wo.reshape(1, d)
    bo2 = bo.reshape(1, d)
    row_spec = pl.BlockSpec((br, d), lambda i: (i, 0))
    vec_spec = pl.BlockSpec((1, d), lambda i: (0, 0))
    return pl.pallas_call(
        _tc_body,
        grid=(n // br,),
        in_specs=[row_spec, row_spec, vec_spec, vec_spec, vec_spec, vec_spec],
        out_specs=[row_spec, row_spec, pl.BlockSpec((br, 1), lambda i: (i, 0))],
        out_shape=[
            jax.ShapeDtypeStruct((n, d), jnp.float32),
            jax.ShapeDtypeStruct((n, d), jnp.float32),
            jax.ShapeDtypeStruct((n, 1), jnp.float32),
        ],
    )(x_c, x_delta, wb2, bb2, wo2, bo2)


def _sc_call(x_c, x_delta, wb, bb, wo, bo):
    n, d = x_c.shape
    rows_per_w = n // _NW
    chunks = rows_per_w // _R
    pairs = chunks // 2
    jvec = d // _L

    mesh = plsc.VectorSubcoreMesh(core_axis_name="c", subcore_axis_name="s")

    @functools.partial(
        pl.kernel,
        mesh=mesh,
        out_type=[
            jax.ShapeDtypeStruct((n, d), jnp.float32),
            jax.ShapeDtypeStruct((n, d), jnp.float32),
            jax.ShapeDtypeStruct((n,), jnp.float32),
        ],
        scratch_types=[
            pltpu.VMEM((2, _R, d), jnp.float32),
            pltpu.VMEM((2, _R, d), jnp.float32),
            pltpu.VMEM((2, _R, d), jnp.float32),
            pltpu.VMEM((2, _R, d), jnp.float32),
            pltpu.VMEM((2, _R), jnp.float32),
            pltpu.VMEM((d,), jnp.float32),
            pltpu.VMEM((d,), jnp.float32),
            pltpu.VMEM((d,), jnp.float32),
            pltpu.VMEM((d,), jnp.float32),
            pltpu.VMEM((d,), jnp.float32),
            pltpu.VMEM((d,), jnp.float32),
            pltpu.VMEM((d,), jnp.float32),
            pltpu.SemaphoreType.DMA,
            pltpu.SemaphoreType.DMA,
            pltpu.SemaphoreType.DMA,
            pltpu.SemaphoreType.DMA,
        ],
    )
    def k(xc_hbm, xd_hbm, wb_hbm, bb_hbm, wo_hbm, bo_hbm,
          oc_hbm, od_hbm, oa_hbm,
          xc_v, xd_v, oc_v, od_v, oa_v,
          wb_v, bb_v, wo_v, bo_v, awb_v, dw_v, db_v,
          in_s0, in_s1, out_s0, out_s1):
        wid = lax.axis_index("s") * _NC + lax.axis_index("c")
        base = wid * rows_per_w
        in_sems = (in_s0, in_s1)
        out_sems = (out_s0, out_s1)

        pltpu.sync_copy(wb_hbm, wb_v)
        pltpu.sync_copy(bb_hbm, bb_v)
        pltpu.sync_copy(wo_hbm, wo_v)
        pltpu.sync_copy(bo_hbm, bo_v)
        for j in range(jvec):
            sl = pl.ds(j * _L, _L)
            awb_v[sl] = jnp.abs(wb_v[sl])
            dw_v[sl] = wb_v[sl] - wo_v[sl]
            db_v[sl] = bb_v[sl] - bo_v[sl]
        head = pl.ds(0, _L)
        wb0 = wb_v[head][0]
        bb0 = bb_v[head][0]
        wo0 = wo_v[head][0]
        bo0 = bo_v[head][0]
        awb0 = awb_v[head][0]
        lane = lax.iota(jnp.int32, _L)
        mask0 = lane == 0

        def start_in(s, c):
            cb = base + c * _R
            pltpu.async_copy(xc_hbm.at[pl.ds(cb, _R)], xc_v.at[s], in_sems[s])
            pltpu.async_copy(xd_hbm.at[pl.ds(cb, _R)], xd_v.at[s], in_sems[s])

        def wait_in(s, c):
            cb = base + c * _R
            pltpu.make_async_copy(
                xc_hbm.at[pl.ds(cb, _R)], xc_v.at[s], in_sems[s]).wait()
            pltpu.make_async_copy(
                xd_hbm.at[pl.ds(cb, _R)], xd_v.at[s], in_sems[s]).wait()

        def start_out(s, c):
            cb = base + c * _R
            pltpu.async_copy(oc_v.at[s], oc_hbm.at[pl.ds(cb, _R)], out_sems[s])
            pltpu.async_copy(od_v.at[s], od_hbm.at[pl.ds(cb, _R)], out_sems[s])
            pltpu.async_copy(oa_v.at[s], oa_hbm.at[pl.ds(cb, _R)], out_sems[s])

        def wait_out(s, c):
            cb = base + c * _R
            pltpu.make_async_copy(
                oc_v.at[s], oc_hbm.at[pl.ds(cb, _R)], out_sems[s]).wait()
            pltpu.make_async_copy(
                od_v.at[s], od_hbm.at[pl.ds(cb, _R)], out_sems[s]).wait()
            pltpu.make_async_copy(
                oa_v.at[s], oa_hbm.at[pl.ds(cb, _R)], out_sems[s]).wait()

        def row_coeffs(s, r):
            xcv0 = xc_v[s, r, head]
            xdv0 = xd_v[s, r, head]
            t_c = jnp.full((_L,), xcv0[0])
            t_d = jnp.full((_L,), xdv0[0])
            lo = t_c - t_d
            hi = t_c + t_d
            frac = jnp.minimum(1.0, (0.0 - lo) / ((hi - lo) + _EPS))
            a1 = jnp.where(hi <= 0.0, 1.0, jnp.where(lo > 0.0, 0.0, frac))
            a2 = 1.0 - a1
            amax = jnp.maximum(a1, a2)
            rcm = 1.0 / (amax + _EPS)
            ap1 = a1 * rcm
            ap2 = a2 * rcm
            sinv = 1.0 / (a1 + a2 + _EPS)
            u1 = a1 * sinv
            q11 = (ap1 + (1.0 - ap1) * u1) * 0.5
            q21 = ((1.0 - ap2) * u1) * 0.5
            ap1h = ap1 * 0.5
            ap2h = ap2 * 0.5

            upper_l = jnp.minimum(hi, 0.0)
            cL = (lo + upper_l) * 0.5
            dL = (upper_l - lo) * 0.5
            lower_r = jnp.maximum(lo, 0.0)
            cR = (lower_r + hi) * 0.5
            c1_0 = cL * wb0 + bb0
            d1_0 = dL * awb0
            c2_0 = cR * wo0 + bo0
            e0 = c1_0 - c2_0
            g1_0 = ap1h * d1_0
            g2_0 = ap2h * c2_0
            mn0 = jnp.minimum(q11 * e0 - g1_0, q21 * e0 - g2_0)
            mx0 = jnp.maximum(q11 * e0 + g1_0, q21 * e0 + g2_0)
            oc0 = c2_0 + mn0 + mx0
            od0 = mx0 - mn0
            alpha = jnp.minimum(1.0, a1 + a2)
            return (q11, q21, ap1h, ap2h, oc0, od0, alpha)

        def compute_chunk(s):
            z = jnp.zeros((_L,), jnp.float32)

            def do_pair(p, accs):
                acc0, acc1 = accs
                r0 = p * 2
                r1 = r0 + 1
                cfs = (row_coeffs(s, r0), row_coeffs(s, r1))
                for r, cf in ((r0, cfs[0]), (r1, cfs[1])):
                    acc0 = jnp.where(lane == r, cf[6], acc0)
                    acc1 = jnp.where(lane == (r - _L), cf[6], acc1)
                def ld(j):
                    wsl = pl.ds(j * _L, _L)
                    return (wsl,
                            dw_v[wsl], db_v[wsl], wo_v[wsl],
                            bo_v[wsl], awb_v[wsl],
                            (xc_v[s, r0, wsl], xd_v[s, r0, wsl]),
                            (xc_v[s, r1, wsl], xd_v[s, r1, wsl]))

                pend = [ld(0), ld(1)]
                for j in range(jvec):
                    if j + 2 < jvec:
                        pend.append(ld(j + 2))
                    wsl, dwj, dbj, woj, boj, awbj, x0, x1 = pend[j]
                    for (xcv, xdv), cf, r in ((x0, cfs[0], r0),
                                              (x1, cfs[1], r1)):
                        q11, q21, ap1h, ap2h, oc0, od0, _ = cf
                        c2 = xcv * woj + boj
                        e = xcv * dwj + dbj
                        g1 = xdv * awbj * ap1h
                        g2 = ap2h * c2
                        t1 = q11 * e
                        t2 = q21 * e
                        mn = jnp.minimum(t1 - g1, t2 - g2)
                        mx = jnp.maximum(t1 + g1, t2 + g2)
                        occ = c2 + mn + mx
                        odd = mx - mn
                        if j == 0:
                            occ = jnp.where(mask0, oc0, occ)
                            odd = jnp.where(mask0, od0, odd)
                        oc_v[s, r, wsl] = occ
                        od_v[s, r, wsl] = odd
                return (acc0, acc1)

            acc0, acc1 = plsc.parallel_loop(
                0, _R // 2, unroll=1, carry=(z, z))(do_pair)
            oa_v[s, pl.ds(0, _L)] = acc0
            oa_v[s, pl.ds(_L, _L)] = acc1

        start_in(0, 0)
        start_in(1, 1)

        def do_pair_of_chunks(g2, carry):
            for s in (0, 1):
                c = g2 * 2 + s
                wait_in(s, c)

                @pl.when(g2 > 0)
                def _():
                    wait_out(s, c - 2)

                compute_chunk(s)
                start_out(s, c)

                @pl.when(c + 2 < chunks)
                def _():
                    start_in(s, c + 2)
            return carry

        lax.fori_loop(0, pairs, do_pair_of_chunks, 0)
        wait_out(0, chunks - 2)
        wait_out(1, chunks - 1)

    return k(x_c, x_delta, wb, bb, wo, bo)


def kernel(x_c, x_delta, w_body, b_body, w_orelse, b_orelse):
    n, d = x_c.shape
    oc, od, oa = _sc_call(x_c, x_delta, w_body, b_body, w_orelse, b_orelse)
    return oc, od, oa.reshape(n, 1)


# speedup vs baseline: 1.6576x; 1.0274x over previous
import functools

import jax
import jax.numpy as jnp
from jax import lax
from jax.experimental import pallas as pl
from jax.experimental.pallas import tpu as pltpu
from jax.experimental.pallas import tpu_sc as plsc

_EPS = 1e-12
_L = 16
_NC = 2
_NS = 16
_NW = _NC * _NS
_R = 32


def _tc_body(xc_ref, xd_ref, wb_ref, bb_ref, wo_ref, bo_ref,
             c_ref, d_ref, a_ref):
    xc = xc_ref[...]
    xd = xd_ref[...]
    wb = wb_ref[...]
    bb = bb_ref[...]
    wo = wo_ref[...]
    bo = bo_ref[...]

    t_c = xc[:, 0:1]
    t_d = xd[:, 0:1]
    lo = t_c - t_d
    hi = t_c + t_d
    frac = jnp.minimum(1.0, (0.0 - lo) / ((hi - lo) + _EPS))
    a1 = jnp.where(hi <= 0.0, 1.0, jnp.where(lo > 0.0, 0.0, frac))
    a2 = 1.0 - a1

    upper_l = jnp.minimum(hi, 0.0)
    cL = (lo + upper_l) * 0.5
    dL = (upper_l - lo) * 0.5
    lower_r = jnp.maximum(lo, 0.0)
    cR = (lower_r + hi) * 0.5
    dR = (hi - lower_r) * 0.5

    col = lax.broadcasted_iota(jnp.int32, xc.shape, 1)
    is0 = col == 0
    xl_c = jnp.where(is0, cL, xc)
    xl_d = jnp.where(is0, dL, xd)
    xr_c = jnp.where(is0, cR, xc)

    c1 = xl_c * wb + bb
    d1 = xl_d * jnp.abs(wb)
    c2 = xr_c * wo + bo

    amax = jnp.maximum(a1, a2)
    rcm = 1.0 / (amax + _EPS)
    ap1 = a1 * rcm
    ap2 = a2 * rcm
    c_out = (a1 * c1 + a2 * c2) / (a1 + a2 + _EPS)
    nc1 = ap1 * c1 + (1.0 - ap1) * c_out
    nc2 = ap2 * c2 + (1.0 - ap2) * c_out
    nd1 = ap1 * d1
    nd2 = ap2 * c2
    nl = jnp.minimum(nc1 - nd1, nc2 - nd2)
    nr = jnp.maximum(nc1 + nd1, nc2 + nd2)

    c_ref[...] = (nl + nr) * 0.5
    d_ref[...] = (nr - nl) * 0.5
    a_ref[...] = jnp.minimum(1.0, a1 + a2)


def _tc_call(x_c, x_delta, wb, bb, wo, bo):
    n, d = x_c.shape
    br = 1024
    wb2 = wb.reshape(1, d)
    bb2 = bb.reshape(1, d)
    wo2 = wo.reshape(1, d)
    bo2 = bo.reshape(1, d)
    row_spec = pl.BlockSpec((br, d), lambda i: (i, 0))
    vec_spec = pl.BlockSpec((1, d), lambda i: (0, 0))
    return pl.pallas_call(
        _tc_body,
        grid=(n // br,),
        in_specs=[row_spec, row_spec, vec_spec, vec_spec, vec_spec, vec_spec],
        out_specs=[row_spec, row_spec, pl.BlockSpec((br, 1), lambda i: (i, 0))],
        out_shape=[
            jax.ShapeDtypeStruct((n, d), jnp.float32),
            jax.ShapeDtypeStruct((n, d), jnp.float32),
            jax.ShapeDtypeStruct((n, 1), jnp.float32),
        ],
    )(x_c, x_delta, wb2, bb2, wo2, bo2)


def _sc_call(x_c, x_delta, wb, bb, wo, bo):
    n, d = x_c.shape
    rows_per_w = n // _NW
    chunks = rows_per_w // _R
    pairs = chunks // 2
    jvec = d // _L

    mesh = plsc.VectorSubcoreMesh(core_axis_name="c", subcore_axis_name="s")

    @functools.partial(
        pl.kernel,
        mesh=mesh,
        out_type=[
            jax.ShapeDtypeStruct((n, d), jnp.float32),
            jax.ShapeDtypeStruct((n, d), jnp.float32),
            jax.ShapeDtypeStruct((n,), jnp.float32),
        ],
        scratch_types=[
            pltpu.VMEM((2, _R, d), jnp.float32),
            pltpu.VMEM((2, _R, d), jnp.float32),
            pltpu.VMEM((2, _R, d), jnp.float32),
            pltpu.VMEM((2, _R, d), jnp.float32),
            pltpu.VMEM((2, _R), jnp.float32),
            pltpu.VMEM((d,), jnp.float32),
            pltpu.VMEM((d,), jnp.float32),
            pltpu.VMEM((d,), jnp.float32),
            pltpu.VMEM((d,), jnp.float32),
            pltpu.VMEM((d,), jnp.float32),
            pltpu.VMEM((d,), jnp.float32),
            pltpu.VMEM((d,), jnp.float32),
            pltpu.SemaphoreType.DMA,
            pltpu.SemaphoreType.DMA,
            pltpu.SemaphoreType.DMA,
            pltpu.SemaphoreType.DMA,
        ],
    )
    def k(xc_hbm, xd_hbm, wb_hbm, bb_hbm, wo_hbm, bo_hbm,
          oc_hbm, od_hbm, oa_hbm,
          xc_v, xd_v, oc_v, od_v, oa_v,
          wb_v, bb_v, wo_v, bo_v, awb_v, dw_v, db_v,
          in_s0, in_s1, out_s0, out_s1):
        wid = lax.axis_index("s") * _NC + lax.axis_index("c")
        base = wid * rows_per_w
        in_sems = (in_s0, in_s1)
        out_sems = (out_s0, out_s1)

        pltpu.sync_copy(wb_hbm, wb_v)
        pltpu.sync_copy(bb_hbm, bb_v)
        pltpu.sync_copy(wo_hbm, wo_v)
        pltpu.sync_copy(bo_hbm, bo_v)
        for j in range(jvec):
            sl = pl.ds(j * _L, _L)
            awb_v[sl] = jnp.abs(wb_v[sl])
            dw_v[sl] = wb_v[sl] - wo_v[sl]
            db_v[sl] = bb_v[sl] - bo_v[sl]
        head = pl.ds(0, _L)
        wb0 = wb_v[head][0]
        bb0 = bb_v[head][0]
        wo0 = wo_v[head][0]
        bo0 = bo_v[head][0]
        awb0 = awb_v[head][0]
        lane = lax.iota(jnp.int32, _L)
        mask0 = lane == 0

        def start_in(s, c):
            cb = base + c * _R
            pltpu.async_copy(xc_hbm.at[pl.ds(cb, _R)], xc_v.at[s], in_sems[s])
            pltpu.async_copy(xd_hbm.at[pl.ds(cb, _R)], xd_v.at[s], in_sems[s])

        def wait_in(s, c):
            cb = base + c * _R
            pltpu.make_async_copy(
                xc_hbm.at[pl.ds(cb, _R)], xc_v.at[s], in_sems[s]).wait()
            pltpu.make_async_copy(
                xd_hbm.at[pl.ds(cb, _R)], xd_v.at[s], in_sems[s]).wait()

        def start_out(s, c):
            cb = base + c * _R
            pltpu.async_copy(oc_v.at[s], oc_hbm.at[pl.ds(cb, _R)], out_sems[s])
            pltpu.async_copy(od_v.at[s], od_hbm.at[pl.ds(cb, _R)], out_sems[s])
            pltpu.async_copy(oa_v.at[s], oa_hbm.at[pl.ds(cb, _R)], out_sems[s])

        def wait_out(s, c):
            cb = base + c * _R
            pltpu.make_async_copy(
                oc_v.at[s], oc_hbm.at[pl.ds(cb, _R)], out_sems[s]).wait()
            pltpu.make_async_copy(
                od_v.at[s], od_hbm.at[pl.ds(cb, _R)], out_sems[s]).wait()
            pltpu.make_async_copy(
                oa_v.at[s], oa_hbm.at[pl.ds(cb, _R)], out_sems[s]).wait()

        def row_coeffs(s, r):
            xcv0 = xc_v[s, r, head]
            xdv0 = xd_v[s, r, head]
            t_c = jnp.full((_L,), xcv0[0])
            t_d = jnp.full((_L,), xdv0[0])
            lo = t_c - t_d
            hi = t_c + t_d
            frac = jnp.minimum(1.0, (0.0 - lo) / ((hi - lo) + _EPS))
            a1 = jnp.where(hi <= 0.0, 1.0, jnp.where(lo > 0.0, 0.0, frac))
            a2 = 1.0 - a1
            amax = jnp.maximum(a1, a2)
            rcm = 1.0 / (amax + _EPS)
            ap1 = a1 * rcm
            ap2 = a2 * rcm
            sinv = 1.0 / (a1 + a2 + _EPS)
            u1 = a1 * sinv
            q11 = (ap1 + (1.0 - ap1) * u1) * 0.5
            q21 = ((1.0 - ap2) * u1) * 0.5
            ap1h = ap1 * 0.5
            ap2h = ap2 * 0.5

            upper_l = jnp.minimum(hi, 0.0)
            cL = (lo + upper_l) * 0.5
            dL = (upper_l - lo) * 0.5
            lower_r = jnp.maximum(lo, 0.0)
            cR = (lower_r + hi) * 0.5
            c1_0 = cL * wb0 + bb0
            d1_0 = dL * awb0
            c2_0 = cR * wo0 + bo0
            e0 = c1_0 - c2_0
            g1_0 = ap1h * d1_0
            g2_0 = ap2h * c2_0
            mn0 = jnp.minimum(q11 * e0 - g1_0, q21 * e0 - g2_0)
            mx0 = jnp.maximum(q11 * e0 + g1_0, q21 * e0 + g2_0)
            oc0 = c2_0 + mn0 + mx0
            od0 = mx0 - mn0
            alpha = jnp.minimum(1.0, a1 + a2)
            return (q11, q21, ap1h, ap2h, oc0, od0, alpha)

        def compute_chunk(s):
            z = jnp.zeros((_L,), jnp.float32)

            nrow = 4

            def do_quad(p, accs):
                acc0, acc1 = accs
                rows = [p * nrow + i for i in range(nrow)]
                cfs = [row_coeffs(s, r) for r in rows]
                for r, cf in zip(rows, cfs):
                    acc0 = jnp.where(lane == r, cf[6], acc0)
                    acc1 = jnp.where(lane == (r - _L), cf[6], acc1)
                def ld(j):
                    wsl = pl.ds(j * _L, _L)
                    return (wsl,
                            dw_v[wsl], db_v[wsl], wo_v[wsl],
                            bo_v[wsl], awb_v[wsl],
                            [(xc_v[s, r, wsl], xd_v[s, r, wsl])
                             for r in rows])

                pend = [ld(0)]
                for j in range(jvec):
                    if j + 1 < jvec:
                        pend.append(ld(j + 1))
                    wsl, dwj, dbj, woj, boj, awbj, xs = pend[j]
                    for (xcv, xdv), cf, r in zip(xs, cfs, rows):
                        q11, q21, ap1h, ap2h, oc0, od0, _ = cf
                        c2 = xcv * woj + boj
                        e = xcv * dwj + dbj
                        g1 = xdv * awbj * ap1h
                        g2 = ap2h * c2
                        t1 = q11 * e
                        t2 = q21 * e
                        mn = jnp.minimum(t1 - g1, t2 - g2)
                        mx = jnp.maximum(t1 + g1, t2 + g2)
                        occ = c2 + mn + mx
                        odd = mx - mn
                        if j == 0:
                            occ = jnp.where(mask0, oc0, occ)
                            odd = jnp.where(mask0, od0, odd)
                        oc_v[s, r, wsl] = occ
                        od_v[s, r, wsl] = odd
                return (acc0, acc1)

            acc0, acc1 = plsc.parallel_loop(
                0, _R // nrow, unroll=1, carry=(z, z))(do_quad)
            oa_v[s, pl.ds(0, _L)] = acc0
            oa_v[s, pl.ds(_L, _L)] = acc1

        start_in(0, 0)
        start_in(1, 1)

        def do_pair_of_chunks(g2, carry):
            for s in (0, 1):
                c = g2 * 2 + s
                wait_in(s, c)

                @pl.when(g2 > 0)
                def _():
                    wait_out(s, c - 2)

                compute_chunk(s)
                start_out(s, c)

                @pl.when(c + 2 < chunks)
                def _():
                    start_in(s, c + 2)
            return carry

        lax.fori_loop(0, pairs, do_pair_of_chunks, 0)
        wait_out(0, chunks - 2)
        wait_out(1, chunks - 1)

    return k(x_c, x_delta, wb, bb, wo, bo)


def kernel(x_c, x_delta, w_body, b_body, w_orelse, b_orelse):
    n, d = x_c.shape
    oc, od, oa = _sc_call(x_c, x_delta, w_body, b_body, w_orelse, b_orelse)
    return oc, od, oa.reshape(n, 1)


# speedup vs baseline: 1.6581x; 1.0003x over previous
import functools

import jax
import jax.numpy as jnp
from jax import lax
from jax.experimental import pallas as pl
from jax.experimental.pallas import tpu as pltpu
from jax.experimental.pallas import tpu_sc as plsc

_EPS = 1e-12
_L = 16
_NC = 2
_NS = 16
_NW = _NC * _NS
_R = 32


def _sc_call(x_c, x_delta, wb, bb, wo, bo):
    n, d = x_c.shape
    rows_per_w = n // _NW
    chunks = rows_per_w // _R
    pairs = chunks // 2
    jvec = d // _L

    mesh = plsc.VectorSubcoreMesh(core_axis_name="c", subcore_axis_name="s")

    @functools.partial(
        pl.kernel,
        mesh=mesh,
        out_type=[
            jax.ShapeDtypeStruct((n, d), jnp.float32),
            jax.ShapeDtypeStruct((n, d), jnp.float32),
            jax.ShapeDtypeStruct((n,), jnp.float32),
        ],
        scratch_types=[
            pltpu.VMEM((2, _R, d), jnp.float32),
            pltpu.VMEM((2, _R, d), jnp.float32),
            pltpu.VMEM((2, _R, d), jnp.float32),
            pltpu.VMEM((2, _R, d), jnp.float32),
            pltpu.VMEM((2, _R), jnp.float32),
            pltpu.VMEM((d,), jnp.float32),
            pltpu.VMEM((d,), jnp.float32),
            pltpu.VMEM((d,), jnp.float32),
            pltpu.VMEM((d,), jnp.float32),
            pltpu.VMEM((d,), jnp.float32),
            pltpu.VMEM((d,), jnp.float32),
            pltpu.VMEM((d,), jnp.float32),
            pltpu.SemaphoreType.DMA,
            pltpu.SemaphoreType.DMA,
            pltpu.SemaphoreType.DMA,
            pltpu.SemaphoreType.DMA,
        ],
    )
    def k(xc_hbm, xd_hbm, wb_hbm, bb_hbm, wo_hbm, bo_hbm,
          oc_hbm, od_hbm, oa_hbm,
          xc_v, xd_v, oc_v, od_v, oa_v,
          wb_v, bb_v, wo_v, bo_v, awb_v, dw_v, db_v,
          in_s0, in_s1, out_s0, out_s1):
        wid = lax.axis_index("s") * _NC + lax.axis_index("c")
        base = wid * rows_per_w
        in_sems = (in_s0, in_s1)
        out_sems = (out_s0, out_s1)

        pltpu.sync_copy(wb_hbm, wb_v)
        pltpu.sync_copy(bb_hbm, bb_v)
        pltpu.sync_copy(wo_hbm, wo_v)
        pltpu.sync_copy(bo_hbm, bo_v)
        for j in range(jvec):
            sl = pl.ds(j * _L, _L)
            awb_v[sl] = jnp.abs(wb_v[sl])
            dw_v[sl] = wb_v[sl] - wo_v[sl]
            db_v[sl] = bb_v[sl] - bo_v[sl]
        head = pl.ds(0, _L)
        wb0 = wb_v[head][0]
        bb0 = bb_v[head][0]
        wo0 = wo_v[head][0]
        bo0 = bo_v[head][0]
        awb0 = awb_v[head][0]
        lane = lax.iota(jnp.int32, _L)
        mask0 = lane == 0

        def start_in(s, c):
            cb = base + c * _R
            pltpu.async_copy(xc_hbm.at[pl.ds(cb, _R)], xc_v.at[s], in_sems[s])
            pltpu.async_copy(xd_hbm.at[pl.ds(cb, _R)], xd_v.at[s], in_sems[s])

        def wait_in(s, c):
            cb = base + c * _R
            pltpu.make_async_copy(
                xc_hbm.at[pl.ds(cb, _R)], xc_v.at[s], in_sems[s]).wait()
            pltpu.make_async_copy(
                xd_hbm.at[pl.ds(cb, _R)], xd_v.at[s], in_sems[s]).wait()

        def start_out(s, c):
            cb = base + c * _R
            pltpu.async_copy(oc_v.at[s], oc_hbm.at[pl.ds(cb, _R)], out_sems[s])
            pltpu.async_copy(od_v.at[s], od_hbm.at[pl.ds(cb, _R)], out_sems[s])
            pltpu.async_copy(oa_v.at[s], oa_hbm.at[pl.ds(cb, _R)], out_sems[s])

        def wait_out(s, c):
            cb = base + c * _R
            pltpu.make_async_copy(
                oc_v.at[s], oc_hbm.at[pl.ds(cb, _R)], out_sems[s]).wait()
            pltpu.make_async_copy(
                od_v.at[s], od_hbm.at[pl.ds(cb, _R)], out_sems[s]).wait()
            pltpu.make_async_copy(
                oa_v.at[s], oa_hbm.at[pl.ds(cb, _R)], out_sems[s]).wait()

        def row_coeffs(s, r):
            xcv0 = xc_v[s, r, head]
            xdv0 = xd_v[s, r, head]
            t_c = jnp.full((_L,), xcv0[0])
            t_d = jnp.full((_L,), xdv0[0])
            lo = t_c - t_d
            hi = t_c + t_d
            frac = jnp.minimum(1.0, (0.0 - lo) / ((hi - lo) + _EPS))
            a1 = jnp.where(hi <= 0.0, 1.0, jnp.where(lo > 0.0, 0.0, frac))
            a2 = 1.0 - a1
            amax = jnp.maximum(a1, a2)
            rcm = 1.0 / (amax + _EPS)
            ap1 = a1 * rcm
            ap2 = a2 * rcm
            sinv = 1.0 / (a1 + a2 + _EPS)
            u1 = a1 * sinv
            q11 = (ap1 + (1.0 - ap1) * u1) * 0.5
            q21 = ((1.0 - ap2) * u1) * 0.5
            ap1h = ap1 * 0.5
            ap2h = ap2 * 0.5

            upper_l = jnp.minimum(hi, 0.0)
            cL = (lo + upper_l) * 0.5
            dL = (upper_l - lo) * 0.5
            lower_r = jnp.maximum(lo, 0.0)
            cR = (lower_r + hi) * 0.5
            c1_0 = cL * wb0 + bb0
            d1_0 = dL * awb0
            c2_0 = cR * wo0 + bo0
            e0 = c1_0 - c2_0
            g1_0 = ap1h * d1_0
            g2_0 = ap2h * c2_0
            mn0 = jnp.minimum(q11 * e0 - g1_0, q21 * e0 - g2_0)
            mx0 = jnp.maximum(q11 * e0 + g1_0, q21 * e0 + g2_0)
            oc0 = c2_0 + mn0 + mx0
            od0 = mx0 - mn0
            alpha = jnp.minimum(1.0, a1 + a2)
            return (q11, q21, ap1h, ap2h, oc0, od0, alpha)

        def compute_chunk(s):
            z = jnp.zeros((_L,), jnp.float32)

            nrow = 4

            def do_quad(p, accs):
                acc0, acc1 = accs
                rows = [p * nrow + i for i in range(nrow)]
                cfs = [row_coeffs(s, r) for r in rows]
                for r, cf in zip(rows, cfs):
                    acc0 = jnp.where(lane == r, cf[6], acc0)
                    acc1 = jnp.where(lane == (r - _L), cf[6], acc1)
                def ld(j):
                    wsl = pl.ds(j * _L, _L)
                    return (wsl,
                            dw_v[wsl], db_v[wsl], wo_v[wsl],
                            bo_v[wsl], awb_v[wsl],
                            [(xc_v[s, r, wsl], xd_v[s, r, wsl])
                             for r in rows])

                pend = [ld(0)]
                for j in range(jvec):
                    if j + 1 < jvec:
                        pend.append(ld(j + 1))
                    wsl, dwj, dbj, woj, boj, awbj, xs = pend[j]
                    for (xcv, xdv), cf, r in zip(xs, cfs, rows):
                        q11, q21, ap1h, ap2h, oc0, od0, _ = cf
                        c2 = xcv * woj + boj
                        e = xcv * dwj + dbj
                        g1 = xdv * awbj * ap1h
                        g2 = ap2h * c2
                        t1 = q11 * e
                        t2 = q21 * e
                        mn = jnp.minimum(t1 - g1, t2 - g2)
                        mx = jnp.maximum(t1 + g1, t2 + g2)
                        occ = c2 + mn + mx
                        odd = mx - mn
                        if j == 0:
                            occ = jnp.where(mask0, oc0, occ)
                            odd = jnp.where(mask0, od0, odd)
                        oc_v[s, r, wsl] = occ
                        od_v[s, r, wsl] = odd
                return (acc0, acc1)

            acc0, acc1 = plsc.parallel_loop(
                0, _R // nrow, unroll=1, carry=(z, z))(do_quad)
            oa_v[s, pl.ds(0, _L)] = acc0
            oa_v[s, pl.ds(_L, _L)] = acc1

        start_in(0, 0)
        start_in(1, 1)

        def do_pair_of_chunks(g2, carry):
            for s in (0, 1):
                c = g2 * 2 + s
                wait_in(s, c)

                @pl.when(g2 > 0)
                def _():
                    wait_out(s, c - 2)

                compute_chunk(s)
                start_out(s, c)

                @pl.when(c + 2 < chunks)
                def _():
                    start_in(s, c + 2)
            return carry

        lax.fori_loop(0, pairs, do_pair_of_chunks, 0)
        wait_out(0, chunks - 2)
        wait_out(1, chunks - 1)

    return k(x_c, x_delta, wb, bb, wo, bo)


def kernel(x_c, x_delta, w_body, b_body, w_orelse, b_orelse):
    n, d = x_c.shape
    oc, od, oa = _sc_call(x_c, x_delta, w_body, b_body, w_orelse, b_orelse)
    return oc, od, oa.reshape(n, 1)
